# Initial kernel scaffold; baseline (speedup 1.0000x reference)
import jax
import jax.numpy as jnp
from jax.experimental import pallas as pl


def kernel(x, train_edge_index, edge_index, edge_attr, W1, b1, W2, b2, L1w, L1b, L2w, L2b):
    raise NotImplementedError("write your pallas kernel here")


# speedup vs baseline: 17.6991x; 17.6991x over previous
import functools

import jax
import jax.numpy as jnp
from jax import lax
from jax.experimental import pallas as pl
from jax.experimental.pallas import tpu as pltpu
from jax.experimental.pallas import tpu_sc as plsc

N = 10000
NP = 10240
E_TRAIN = 640000
E_DEC = 100000
ED_PAD = 102400
HID = 768
NC = 5
CH = 128
NCHUNK = E_TRAIN // CH
NCHUNK_HALF = NCHUNK // 2
NDCH = ED_PAD // CH
NSUB = 16
SLC = NP // NSUB

_mesh = plsc.VectorSubcoreMesh(core_axis_name="c", subcore_axis_name="s")


def _fill_const(ref, n16, value):
    @pl.loop(0, n16)
    def _(i):
        ref[pl.ds(i * 16, 16)] = jnp.full((16,), value, jnp.float32)


@functools.partial(
    pl.kernel,
    out_type=jax.ShapeDtypeStruct((2, NP), jnp.float32),
    mesh=_mesh,
    compiler_params=pltpu.CompilerParams(needs_layout_passes=False, use_tc_tiling_on_sc=False),
    scratch_types=[
        pltpu.VMEM((CH,), jnp.int32),
        pltpu.VMEM((CH,), jnp.float32),
        pltpu.VMEM((SLC,), jnp.float32),
        pltpu.VMEM_SHARED((NP,), jnp.float32),
    ],
)
def _sc_deg(td2d, deg_out, idx_a, ones_v, fill_v, deg_s):
    c = lax.axis_index("c")
    s = lax.axis_index("s")
    base = s * SLC

    _fill_const(fill_v, SLC // 16, 0.0)
    pltpu.sync_copy(fill_v, deg_s.at[pl.ds(base, SLC)])
    _fill_const(ones_v, CH // 16, 1.0)
    plsc.subcore_barrier()

    @pl.loop(c * NCHUNK_HALF + s, (c + 1) * NCHUNK_HALF, step=NSUB)
    def _(ci):
        pltpu.sync_copy(td2d.at[ci], idx_a)
        pltpu.sync_copy(ones_v, deg_s.at[idx_a], add=True)

    plsc.subcore_barrier()
    pltpu.sync_copy(deg_s.at[pl.ds(base, SLC)], deg_out.at[c, pl.ds(base, SLC)])


@functools.partial(
    pl.kernel,
    out_type=jax.ShapeDtypeStruct((2, NP), jnp.float32),
    mesh=_mesh,
    compiler_params=pltpu.CompilerParams(needs_layout_passes=False, use_tc_tiling_on_sc=False),
    scratch_types=[
        pltpu.VMEM((CH,), jnp.int32),
        pltpu.VMEM((CH,), jnp.int32),
        pltpu.VMEM((CH,), jnp.float32),
        pltpu.VMEM((SLC,), jnp.float32),
        pltpu.VMEM((NP,), jnp.float32),
        pltpu.VMEM_SHARED((NP,), jnp.float32),
    ],
)
def _sc_g1(ts2d, td2d, u_hbm, g1_out, idx_a, idx_b, val_v, fill_v, tab, g1_s):
    c = lax.axis_index("c")
    s = lax.axis_index("s")
    base = s * SLC

    _fill_const(fill_v, SLC // 16, 0.0)
    pltpu.sync_copy(fill_v, g1_s.at[pl.ds(base, SLC)])
    pltpu.sync_copy(u_hbm, tab)
    plsc.subcore_barrier()

    @pl.loop(c * NCHUNK_HALF + s, (c + 1) * NCHUNK_HALF, step=NSUB)
    def _(ci):
        pltpu.sync_copy(ts2d.at[ci], idx_a)
        pltpu.sync_copy(td2d.at[ci], idx_b)

        @pl.loop(0, CH // 16)
        def _(j):
            sl = pl.ds(j * 16, 16)
            val_v[sl] = plsc.load_gather(tab, [idx_a[sl]])

        pltpu.sync_copy(val_v, g1_s.at[idx_b], add=True)

    plsc.subcore_barrier()
    pltpu.sync_copy(g1_s.at[pl.ds(base, SLC)], g1_out.at[c, pl.ds(base, SLC)])


@functools.partial(
    pl.kernel,
    out_type=jax.ShapeDtypeStruct((2, NP, 64), jnp.float32),
    mesh=_mesh,
    compiler_params=pltpu.CompilerParams(needs_layout_passes=False, use_tc_tiling_on_sc=False),
    scratch_types=[
        pltpu.VMEM((CH,), jnp.int32),
        pltpu.VMEM((CH,), jnp.int32),
        pltpu.VMEM((CH, 64), jnp.float32),
        pltpu.VMEM((CH, 64), jnp.float32),
        pltpu.VMEM_SHARED((NP, 64), jnp.float32),
        pltpu.SemaphoreType.DMA,
    ],
)
def _sc_stage2(ts2d, td2d, u2_hbm, g2_out, idx_a, idx_b, rows, zbuf, acc_s, sem):
    c = lax.axis_index("c")
    s = lax.axis_index("s")

    @pl.loop(0, CH)
    def _(r):
        for j in range(4):
            zbuf[r, pl.ds(j * 16, 16)] = jnp.zeros((16,), jnp.float32)

    for k in range(SLC // CH):
        pltpu.sync_copy(zbuf, acc_s.at[pl.ds(s * SLC + k * CH, CH)])
    plsc.subcore_barrier()

    @pl.loop(c * NCHUNK_HALF + s, (c + 1) * NCHUNK_HALF, step=NSUB)
    def _(ci):
        pltpu.sync_copy(ts2d.at[ci], idx_a)
        pltpu.async_copy(u2_hbm.at[idx_a], rows, sem).wait()
        pltpu.sync_copy(td2d.at[ci], idx_b)
        pltpu.sync_copy(rows, acc_s.at[idx_b], add=True)

    plsc.subcore_barrier()
    pltpu.sync_copy(acc_s.at[pl.ds(s * SLC, SLC)],
                    g2_out.at[c, pl.ds(s * SLC, SLC)])


@functools.partial(
    pl.kernel,
    out_type=jax.ShapeDtypeStruct((ED_PAD, 64), jnp.float32),
    mesh=_mesh,
    compiler_params=pltpu.CompilerParams(needs_layout_passes=False, use_tc_tiling_on_sc=False),
    scratch_types=[
        pltpu.VMEM((CH,), jnp.int32),
        pltpu.VMEM((CH,), jnp.int32),
        pltpu.VMEM((CH, 64), jnp.float32),
        pltpu.VMEM((CH, 64), jnp.float32),
        pltpu.SemaphoreType.DMA,
    ],
)
def _sc_stage3(e0_2d, e1_2d, z2_hbm, nr_out, idx_a, idx_b, rows0, rows1, sem):
    c = lax.axis_index("c")
    s = lax.axis_index("s")
    wid = s * 2 + c

    @pl.loop(wid, NDCH, step=32)
    def _(ci):
        pltpu.sync_copy(e0_2d.at[ci], idx_a)
        pltpu.async_copy(z2_hbm.at[idx_a], rows0, sem).wait()
        pltpu.sync_copy(e1_2d.at[ci], idx_b)
        pltpu.async_copy(z2_hbm.at[idx_b], rows1, sem).wait()

        @pl.loop(0, CH)
        def _(r):
            for j in range(4):
                sl = pl.ds(j * 16, 16)
                rows0[r, sl] = rows0[r, sl] * rows1[r, sl]

        pltpu.sync_copy(rows0, nr_out.at[pl.ds(ci * CH, CH)])


def _tc_prep_body(dega_ref, degb_ref, x_ref, dinv_ref, u_ref):
    deg = dega_ref[...] + degb_ref[...] + 1.0
    dinv = lax.rsqrt(jnp.maximum(deg, 1e-12))
    dinv_ref[...] = dinv
    u_ref[...] = x_ref[...] * dinv


def _tc_mid_body(dinv_ref, u_ref, g1a_ref, g1b_ref, W1_ref, b1_ref, W2_ref,
                 u2_ref):
    dinv = dinv_ref[...]
    u = u_ref[...]
    s1 = dinv * (g1a_ref[...] + g1b_ref[...] + u)
    z1 = jnp.maximum(s1 * W1_ref[...] + b1_ref[...], 0.0)
    h2 = jnp.dot(z1, W2_ref[...], preferred_element_type=jnp.float32)
    u2_ref[...] = h2 * dinv


def _tc_z2_body(dinv_ref, g2a_ref, g2b_ref, u2_ref, b2_ref, z2_ref):
    dinv = dinv_ref[...]
    agg = dinv * (g2a_ref[...] + g2b_ref[...] + u2_ref[...])
    z2_ref[...] = jnp.maximum(agg + b2_ref[...], 0.0)


def _tc_dec_body(nr_ref, ea_ref, L1n_ref, L1a_ref, L1b_ref, L2w_ref, L2b_ref,
                 out_ref):
    a = jnp.dot(nr_ref[...], L1n_ref[...], preferred_element_type=jnp.float32)
    a += jnp.dot(ea_ref[...], L1a_ref[...], preferred_element_type=jnp.float32)
    h = jnp.maximum(a + L1b_ref[...], 0.0)
    logits = jnp.dot(h, L2w_ref[...], preferred_element_type=jnp.float32)
    logits += L2b_ref[...]
    m = jnp.max(logits, axis=-1, keepdims=True)
    e = jnp.exp(logits - m)
    out_ref[...] = e / jnp.sum(e, axis=-1, keepdims=True)


_EB = 2048


def kernel(x, train_edge_index, edge_index, edge_attr,
           W1, b1, W2, b2, L1w, L1b, L2w, L2b):
    f32 = jnp.float32
    ts2d = train_edge_index[0].reshape(NCHUNK, CH)
    td2d = train_edge_index[1].reshape(NCHUNK, CH)
    e0_2d = jnp.pad(edge_index[0], (0, ED_PAD - E_DEC)).reshape(NDCH, CH)
    e1_2d = jnp.pad(edge_index[1], (0, ED_PAD - E_DEC)).reshape(NDCH, CH)
    xp = jnp.pad(x[:, 0], (0, NP - N))

    deg = _sc_deg(td2d)

    dinv2, u = pl.pallas_call(
        _tc_prep_body,
        out_shape=(jax.ShapeDtypeStruct((NP, 1), f32),
                   jax.ShapeDtypeStruct((NP, 1), f32)),
    )(deg[0].reshape(NP, 1), deg[1].reshape(NP, 1), xp.reshape(NP, 1))

    g1 = _sc_g1(ts2d, td2d, u.reshape(NP))

    u2 = pl.pallas_call(
        _tc_mid_body,
        out_shape=jax.ShapeDtypeStruct((NP, 64), f32),
    )(dinv2, u, g1[0].reshape(NP, 1), g1[1].reshape(NP, 1),
      W1, b1.reshape(1, 128), W2)

    g2 = _sc_stage2(ts2d, td2d, u2)

    z2 = pl.pallas_call(
        _tc_z2_body,
        out_shape=jax.ShapeDtypeStruct((NP, 64), f32),
    )(dinv2, g2[0], g2[1], u2, b2.reshape(1, 64))

    nr = _sc_stage3(e0_2d, e1_2d, z2)

    grid = (E_DEC + _EB - 1) // _EB
    out = pl.pallas_call(
        _tc_dec_body,
        grid=(grid,),
        in_specs=[
            pl.BlockSpec((_EB, 64), lambda i: (i, 0)),
            pl.BlockSpec((_EB, HID), lambda i: (i, 0)),
            pl.BlockSpec((64, 128), lambda i: (0, 0)),
            pl.BlockSpec((HID, 128), lambda i: (0, 0)),
            pl.BlockSpec((1, 128), lambda i: (0, 0)),
            pl.BlockSpec((128, NC), lambda i: (0, 0)),
            pl.BlockSpec((1, NC), lambda i: (0, 0)),
        ],
        out_specs=pl.BlockSpec((_EB, NC), lambda i: (i, 0)),
        out_shape=jax.ShapeDtypeStruct((E_DEC, NC), f32),
    )(nr, edge_attr, L1w[:64], L1w[64:], L1b.reshape(1, 128),
      L2w, L2b.reshape(1, NC))

    return out


# speedup vs baseline: 19.1610x; 1.0826x over previous
import functools

import jax
import jax.numpy as jnp
from jax import lax
from jax.experimental import pallas as pl
from jax.experimental.pallas import tpu as pltpu
from jax.experimental.pallas import tpu_sc as plsc

N = 10000
NP = 10240
E_TRAIN = 640000
E_PAD = 655360
E_DEC = 100000
ED_PAD = 102400
HID = 768
NC = 5
CH = 128
NCHUNK = E_PAD // CH
NCHUNK_HALF = NCHUNK // 2
TCH = NCHUNK_HALF // 16
NDCH = ED_PAD // CH
DCH_W = NDCH // 32
NSUB = 16
SLC = NP // NSUB

_mesh = plsc.VectorSubcoreMesh(core_axis_name="c", subcore_axis_name="s")
_sc_params = pltpu.CompilerParams(needs_layout_passes=False,
                                  use_tc_tiling_on_sc=False)


def _fill_const(ref, n16, value):
    @pl.loop(0, n16)
    def _(i):
        ref[pl.ds(i * 16, 16)] = jnp.full((16,), value, jnp.float32)


@functools.partial(
    pl.kernel,
    out_type=jax.ShapeDtypeStruct((2, NP), jnp.float32),
    mesh=_mesh,
    compiler_params=_sc_params,
    scratch_types=[
        pltpu.VMEM((8, CH), jnp.int32),
        pltpu.VMEM((CH,), jnp.float32),
        pltpu.VMEM((SLC,), jnp.float32),
        pltpu.VMEM_SHARED((NP,), jnp.float32),
        pltpu.SemaphoreType.DMA,
    ],
)
def _sc_deg(td2d, deg_out, idx2, ones_v, fill_v, deg_s, sem):
    c = lax.axis_index("c")
    s = lax.axis_index("s")
    base = s * SLC
    start = c * NCHUNK_HALF + s * TCH

    _fill_const(fill_v, SLC // 16, 0.0)
    pltpu.sync_copy(fill_v, deg_s.at[pl.ds(base, SLC)])
    _fill_const(ones_v, CH // 16, 1.0)
    plsc.subcore_barrier()

    @pl.loop(0, TCH // 8)
    def _(b):
        cb = start + b * 8
        pltpu.sync_copy(td2d.at[pl.ds(cb, 8)], idx2)
        descs = [pltpu.async_copy(ones_v, deg_s.at[idx2.at[j]], sem, add=True)
                 for j in range(8)]
        for d in descs:
            d.wait()

    plsc.subcore_barrier()
    pltpu.sync_copy(deg_s.at[pl.ds(base, SLC)], deg_out.at[c, pl.ds(base, SLC)])


@functools.partial(
    pl.kernel,
    out_type=jax.ShapeDtypeStruct((2, NP), jnp.float32),
    mesh=_mesh,
    compiler_params=_sc_params,
    scratch_types=[
        pltpu.VMEM((8, CH), jnp.int32),
        pltpu.VMEM((8, CH), jnp.int32),
        pltpu.VMEM((8, CH), jnp.float32),
        pltpu.VMEM((SLC,), jnp.float32),
        pltpu.VMEM((NP,), jnp.float32),
        pltpu.VMEM_SHARED((NP,), jnp.float32),
        pltpu.SemaphoreType.DMA,
    ],
)
def _sc_g1(ts2d, td2d, u_hbm, g1_out, idxa2, idxb2, valb, fill_v, tab, g1_s,
           sem):
    c = lax.axis_index("c")
    s = lax.axis_index("s")
    base = s * SLC
    start = c * NCHUNK_HALF + s * TCH

    _fill_const(fill_v, SLC // 16, 0.0)
    pltpu.sync_copy(fill_v, g1_s.at[pl.ds(base, SLC)])
    pltpu.sync_copy(u_hbm, tab)
    plsc.subcore_barrier()

    @pl.loop(0, TCH // 8)
    def _(b):
        cb = start + b * 8
        pltpu.sync_copy(ts2d.at[pl.ds(cb, 8)], idxa2)
        pltpu.sync_copy(td2d.at[pl.ds(cb, 8)], idxb2)

        @pl.loop(0, 8)
        def _(r):
            for k in range(CH // 16):
                sl = pl.ds(k * 16, 16)
                valb[r, sl] = plsc.load_gather(tab, [idxa2[r, sl]])

        descs = [pltpu.async_copy(valb.at[j], g1_s.at[idxb2.at[j]], sem,
                                  add=True)
                 for j in range(8)]
        for d in descs:
            d.wait()

    plsc.subcore_barrier()
    pltpu.sync_copy(g1_s.at[pl.ds(base, SLC)], g1_out.at[c, pl.ds(base, SLC)])


@functools.partial(
    pl.kernel,
    out_type=jax.ShapeDtypeStruct((2, NP, 64), jnp.float32),
    mesh=_mesh,
    compiler_params=_sc_params,
    scratch_types=[
        pltpu.VMEM((4, CH), jnp.int32),
        pltpu.VMEM((4, CH), jnp.int32),
        pltpu.VMEM((4, CH, 64), jnp.float32),
        pltpu.VMEM((CH, 64), jnp.float32),
        pltpu.VMEM_SHARED((NP, 64), jnp.float32),
        pltpu.SemaphoreType.DMA,
        pltpu.SemaphoreType.DMA,
    ],
)
def _sc_stage2(ts2d, td2d, u2_hbm, g2_out, idxa2, idxb2, rows, zbuf, acc_s,
               gsem, ssem):
    c = lax.axis_index("c")
    s = lax.axis_index("s")
    start = c * NCHUNK_HALF + s * TCH

    @pl.loop(0, CH)
    def _(r):
        for j in range(4):
            zbuf[r, pl.ds(j * 16, 16)] = jnp.zeros((16,), jnp.float32)

    for k in range(SLC // CH):
        pltpu.sync_copy(zbuf, acc_s.at[pl.ds(s * SLC + k * CH, CH)])
    plsc.subcore_barrier()

    @pl.loop(0, TCH // 4)
    def _(b):
        cb = start + b * 4
        pltpu.sync_copy(ts2d.at[pl.ds(cb, 4)], idxa2)
        pltpu.sync_copy(td2d.at[pl.ds(cb, 4)], idxb2)
        gd = [pltpu.async_copy(u2_hbm.at[idxa2.at[j]], rows.at[j], gsem)
              for j in range(4)]
        sd = []
        for j in range(4):
            gd[j].wait()
            sd.append(pltpu.async_copy(rows.at[j], acc_s.at[idxb2.at[j]],
                                       ssem, add=True))
        for d in sd:
            d.wait()

    plsc.subcore_barrier()
    pltpu.sync_copy(acc_s.at[pl.ds(s * SLC, SLC)],
                    g2_out.at[c, pl.ds(s * SLC, SLC)])


@functools.partial(
    pl.kernel,
    out_type=jax.ShapeDtypeStruct((ED_PAD, 64), jnp.float32),
    mesh=_mesh,
    compiler_params=_sc_params,
    scratch_types=[
        pltpu.VMEM((5, CH), jnp.int32),
        pltpu.VMEM((5, CH), jnp.int32),
        pltpu.VMEM((5, CH, 64), jnp.float32),
        pltpu.VMEM((5, CH, 64), jnp.float32),
        pltpu.SemaphoreType.DMA,
        pltpu.SemaphoreType.DMA,
        pltpu.SemaphoreType.DMA,
    ],
)
def _sc_stage3(e0_2d, e1_2d, z2_hbm, nr_out, idxa2, idxb2, rows0, rows1,
               g0sem, g1sem, stsem):
    c = lax.axis_index("c")
    s = lax.axis_index("s")
    wid = s * 2 + c
    start = wid * DCH_W

    @pl.loop(0, DCH_W // 5)
    def _(b):
        cb = start + b * 5
        pltpu.sync_copy(e0_2d.at[pl.ds(cb, 5)], idxa2)
        pltpu.sync_copy(e1_2d.at[pl.ds(cb, 5)], idxb2)
        ga = [pltpu.async_copy(z2_hbm.at[idxa2.at[j]], rows0.at[j], g0sem)
              for j in range(5)]
        gb = [pltpu.async_copy(z2_hbm.at[idxb2.at[j]], rows1.at[j], g1sem)
              for j in range(5)]
        st = []
        for j in range(5):
            ga[j].wait()
            gb[j].wait()

            @pl.loop(0, CH)
            def _(r):
                for k in range(4):
                    sl = pl.ds(k * 16, 16)
                    rows0[j, r, sl] = rows0[j, r, sl] * rows1[j, r, sl]

            st.append(pltpu.async_copy(
                rows0.at[j], nr_out.at[pl.ds((cb + j) * CH, CH)], stsem))
        for d in st:
            d.wait()


def _tc_prep_body(dega_ref, degb_ref, x_ref, dinv_ref, u_ref):
    deg = dega_ref[...] + degb_ref[...] + 1.0
    dinv = lax.rsqrt(jnp.maximum(deg, 1e-12))
    dinv_ref[...] = dinv
    u_ref[...] = x_ref[...] * dinv


def _tc_mid_body(dinv_ref, u_ref, g1a_ref, g1b_ref, W1_ref, b1_ref, W2_ref,
                 u2_ref):
    dinv = dinv_ref[...]
    u = u_ref[...]
    s1 = dinv * (g1a_ref[...] + g1b_ref[...] + u)
    z1 = jnp.maximum(s1 * W1_ref[...] + b1_ref[...], 0.0)
    h2 = jnp.dot(z1, W2_ref[...], preferred_element_type=jnp.float32)
    u2_ref[...] = h2 * dinv


def _tc_z2_body(dinv_ref, g2a_ref, g2b_ref, u2_ref, b2_ref, z2_ref):
    dinv = dinv_ref[...]
    agg = dinv * (g2a_ref[...] + g2b_ref[...] + u2_ref[...])
    z2_ref[...] = jnp.maximum(agg + b2_ref[...], 0.0)


def _tc_dec_body(nr_ref, ea_ref, L1n_ref, L1a_ref, L1b_ref, L2w_ref, L2b_ref,
                 out_ref):
    a = jnp.dot(nr_ref[...], L1n_ref[...], preferred_element_type=jnp.float32)
    a += jnp.dot(ea_ref[...], L1a_ref[...], preferred_element_type=jnp.float32)
    h = jnp.maximum(a + L1b_ref[...], 0.0)
    logits = jnp.dot(h, L2w_ref[...], preferred_element_type=jnp.float32)
    logits += L2b_ref[...]
    m = jnp.max(logits, axis=-1, keepdims=True)
    e = jnp.exp(logits - m)
    out_ref[...] = e / jnp.sum(e, axis=-1, keepdims=True)


_EB = 2048


def kernel(x, train_edge_index, edge_index, edge_attr,
           W1, b1, W2, b2, L1w, L1b, L2w, L2b):
    f32 = jnp.float32
    npad = E_PAD - E_TRAIN
    ts2d = jnp.concatenate(
        [train_edge_index[0],
         jnp.zeros((npad,), jnp.int32)]).reshape(NCHUNK, CH)
    td2d = jnp.concatenate(
        [train_edge_index[1],
         jnp.full((npad,), NP - 1, jnp.int32)]).reshape(NCHUNK, CH)
    e0_2d = jnp.pad(edge_index[0], (0, ED_PAD - E_DEC)).reshape(NDCH, CH)
    e1_2d = jnp.pad(edge_index[1], (0, ED_PAD - E_DEC)).reshape(NDCH, CH)
    xp = jnp.pad(x[:, 0], (0, NP - N))

    deg = _sc_deg(td2d)

    dinv2, u = pl.pallas_call(
        _tc_prep_body,
        out_shape=(jax.ShapeDtypeStruct((NP, 1), f32),
                   jax.ShapeDtypeStruct((NP, 1), f32)),
    )(deg[0].reshape(NP, 1), deg[1].reshape(NP, 1), xp.reshape(NP, 1))

    g1 = _sc_g1(ts2d, td2d, u.reshape(NP))

    u2 = pl.pallas_call(
        _tc_mid_body,
        out_shape=jax.ShapeDtypeStruct((NP, 64), f32),
    )(dinv2, u, g1[0].reshape(NP, 1), g1[1].reshape(NP, 1),
      W1, b1.reshape(1, 128), W2)

    g2 = _sc_stage2(ts2d, td2d, u2)

    z2 = pl.pallas_call(
        _tc_z2_body,
        out_shape=jax.ShapeDtypeStruct((NP, 64), f32),
    )(dinv2, g2[0], g2[1], u2, b2.reshape(1, 64))

    nr = _sc_stage3(e0_2d, e1_2d, z2)

    grid = (E_DEC + _EB - 1) // _EB
    out = pl.pallas_call(
        _tc_dec_body,
        grid=(grid,),
        in_specs=[
            pl.BlockSpec((_EB, 64), lambda i: (i, 0)),
            pl.BlockSpec((_EB, HID), lambda i: (i, 0)),
            pl.BlockSpec((64, 128), lambda i: (0, 0)),
            pl.BlockSpec((HID, 128), lambda i: (0, 0)),
            pl.BlockSpec((1, 128), lambda i: (0, 0)),
            pl.BlockSpec((128, NC), lambda i: (0, 0)),
            pl.BlockSpec((1, NC), lambda i: (0, 0)),
        ],
        out_specs=pl.BlockSpec((_EB, NC), lambda i: (i, 0)),
        out_shape=jax.ShapeDtypeStruct((E_DEC, NC), f32),
    )(nr, edge_attr, L1w[:64], L1w[64:], L1b.reshape(1, 128),
      L2w, L2b.reshape(1, NC))

    return out
